# baseline (speedup 1.0000x reference)
import functools

import jax
import jax.numpy as jnp
from jax import lax
from jax.experimental import pallas as pl
from jax.experimental.pallas import tpu as pltpu
from jax.experimental.pallas import tpu_sc as plsc

N = 160000
B = 8
M = 500
D = 8
NSUB = 16
CHUNK = N // NSUB
PPW = (B * M) // NSUB
PPAD = 256
NV = PPW * D // 16


def _body(output_hbm, bidx_hbm, ind_hbm, mask_hbm, tgt_hbm, out_hbm,
          bi_v, cnt_v, all_v, starts_v, ind_v, mask_v, tgt_v,
          idx_a, idx_b, rows_a, rows_b, part_v, acc_v, out_stage,
          sall_sh, sacc_sh, sem):
    w = lax.axis_index("s")
    iota = lax.iota(jnp.int32, 16)

    pltpu.sync_copy(bidx_hbm.at[pl.ds(w * CHUNK, CHUNK)], bi_v)
    pos = jnp.zeros((16,), jnp.int32)
    step = 8192
    while step:
        npos = pos + step
        probe = jnp.minimum(npos, CHUNK) - 1
        v = plsc.load_gather(bi_v, [probe])
        take = (npos <= CHUNK) & (v < iota)
        pos = jnp.where(take, npos, pos)
        step >>= 1
    cnt_v[...] = pos
    pltpu.sync_copy(cnt_v, sall_sh.at[w])
    plsc.subcore_barrier()
    pltpu.sync_copy(sall_sh, all_v)
    starts = all_v[0]
    for r in range(1, NSUB):
        starts = starts + all_v[r]
    starts_v[...] = starts

    b = w >> 1
    tgt_cp = pltpu.async_copy(tgt_hbm.at[w], tgt_v, sem)
    pltpu.sync_copy(ind_hbm.at[w], ind_v)
    pltpu.sync_copy(mask_hbm.at[w], mask_v)

    start_b = plsc.load_gather(starts_v, [jnp.full((16,), 0, jnp.int32) + b])
    for k in range(8):
        iv = ind_v[pl.ds(k * 16, 16)] + start_b
        idx_a[pl.ds(k * 16, 16)] = jnp.minimum(jnp.maximum(iv, 0), N - 1)
    for k in range(8):
        iv = ind_v[pl.ds(128 + k * 16, 16)] + start_b
        idx_b[pl.ds(k * 16, 16)] = jnp.minimum(jnp.maximum(iv, 0), N - 1)
    cp_a = pltpu.async_copy(output_hbm.at[idx_a], rows_a, sem)
    cp_b = pltpu.async_copy(output_hbm.at[idx_b], rows_b, sem)
    tgt_cp.wait()
    cp_a.wait()
    cp_b.wait()

    ge8 = iota >> 3
    col = iota & 7
    acc = jnp.zeros((16,), jnp.float32)
    for k in range(NV):
        ridx = ge8 + (2 * k)
        if k < 64:
            p = plsc.load_gather(rows_a, [ridx, col])
        else:
            p = plsc.load_gather(rows_b, [ridx - 128, col])
        mrow = plsc.load_gather(mask_v, [ridx])
        t = tgt_v[pl.ds(k * 16, 16)]
        m = mrow * jnp.where(t != t, 0.0, 1.0)
        acc = acc + jnp.abs(p * m - t * m)

    accn = jnp.zeros((16,), jnp.float32)
    for k in range(PPAD // 16):
        accn = accn + mask_v[pl.ds(k * 16, 16)]

    part_v[0] = acc
    part_v[1] = accn
    pltpu.sync_copy(part_v, sacc_sh.at[w])
    plsc.subcore_barrier()

    @pl.when(w == 0)
    def _():
        pltpu.sync_copy(sacc_sh, acc_v)
        lacc = acc_v[0, 0]
        nacc = acc_v[0, 1]
        for r in range(1, NSUB):
            lacc = lacc + acc_v[r, 0]
            nacc = nacc + acc_v[r, 1]
        num = jnp.maximum(jnp.sum(nacc), 1.0)
        out_stage[...] = lacc
        hi = plsc.load_gather(out_stage, [(iota & 7) + 8])
        out_stage[...] = (lacc + hi) / num
        pltpu.sync_copy(out_stage.at[pl.ds(0, D)], out_hbm)


_call = functools.partial(
    pl.kernel,
    out_type=jax.ShapeDtypeStruct((D,), jnp.float32),
    mesh=plsc.VectorSubcoreMesh(core_axis_name="c", subcore_axis_name="s",
                                num_cores=1),
    compiler_params=pltpu.CompilerParams(needs_layout_passes=False,
                                         use_tc_tiling_on_sc=False),
    scratch_types=[
        pltpu.VMEM((CHUNK,), jnp.int32),
        pltpu.VMEM((16,), jnp.int32),
        pltpu.VMEM((NSUB, 16), jnp.int32),
        pltpu.VMEM((16,), jnp.int32),
        pltpu.VMEM((PPAD,), jnp.int32),
        pltpu.VMEM((PPAD,), jnp.float32),
        pltpu.VMEM((PPW * D,), jnp.float32),
        pltpu.VMEM((128,), jnp.int32),
        pltpu.VMEM((128,), jnp.int32),
        pltpu.VMEM((128, D), jnp.float32),
        pltpu.VMEM((128, D), jnp.float32),
        pltpu.VMEM((2, 16), jnp.float32),
        pltpu.VMEM((NSUB, 2, 16), jnp.float32),
        pltpu.VMEM((16,), jnp.float32),
        pltpu.VMEM_SHARED((NSUB, 16), jnp.int32),
        pltpu.VMEM_SHARED((NSUB, 2, 16), jnp.float32),
        pltpu.SemaphoreType.DMA,
    ],
)(_body)


def kernel(output, mask, ind, target, batch_index):
    bidx = batch_index.astype(jnp.int32)
    pad = ((0, 0), (0, PPAD - PPW))
    ind16 = jnp.pad(ind.astype(jnp.int32).reshape(NSUB, PPW), pad)
    mask16 = jnp.pad(mask.reshape(NSUB, PPW).astype(jnp.float32), pad)
    tgt16 = target.reshape(NSUB, PPW * D)
    return _call(output, bidx, ind16, mask16, tgt16)


# speedup vs baseline: 1.0009x; 1.0009x over previous
import functools

import jax
import jax.numpy as jnp
from jax import lax
from jax.experimental import pallas as pl
from jax.experimental.pallas import tpu as pltpu
from jax.experimental.pallas import tpu_sc as plsc

N = 160000
B = 8
M = 500
D = 8
NSUB = 16
CHUNK = N // NSUB
PPW = (B * M) // NSUB
PPAD = 256
NV = PPW * D // 16


def _body(output_hbm, bidx_hbm, ind_hbm, mask_hbm, tgt_hbm, out_hbm,
          bi_v, cnt_v, all_v, starts_v, ind_v, mask_v, tgt_v,
          idx_a, idx_b, rows_a, rows_b, part_v, acc_v, out_stage,
          sall_sh, sacc_sh, sem):
    w = lax.axis_index("s")
    iota = lax.iota(jnp.int32, 16)

    pltpu.sync_copy(bidx_hbm.at[pl.ds(w * CHUNK, CHUNK)], bi_v)
    pos = jnp.zeros((16,), jnp.int32)
    step = 8192
    while step:
        npos = pos + step
        probe = jnp.minimum(npos, CHUNK) - 1
        v = plsc.load_gather(bi_v, [probe])
        take = (npos <= CHUNK) & (v < iota)
        pos = jnp.where(take, npos, pos)
        step >>= 1
    cnt_v[...] = pos
    pltpu.sync_copy(cnt_v, sall_sh.at[w])
    plsc.subcore_barrier()
    pltpu.sync_copy(sall_sh, all_v)
    starts = all_v[0]
    for r in range(1, NSUB):
        starts = starts + all_v[r]
    starts_v[...] = starts

    b = w >> 1
    tgt_cp = pltpu.async_copy(tgt_hbm.at[w], tgt_v, sem)
    pltpu.sync_copy(ind_hbm.at[w], ind_v)
    pltpu.sync_copy(mask_hbm.at[w], mask_v)

    start_b = plsc.load_gather(starts_v, [jnp.full((16,), 0, jnp.int32) + b])
    for k in range(8):
        iv = ind_v[pl.ds(k * 16, 16)] + start_b
        idx_a[pl.ds(k * 16, 16)] = jnp.minimum(jnp.maximum(iv, 0), N - 1)
    for k in range(8):
        iv = ind_v[pl.ds(128 + k * 16, 16)] + start_b
        idx_b[pl.ds(k * 16, 16)] = jnp.minimum(jnp.maximum(iv, 0), N - 1)
    cp_a = pltpu.async_copy(output_hbm.at[idx_a], rows_a, sem)
    cp_b = pltpu.async_copy(output_hbm.at[idx_b], rows_b, sem)
    tgt_cp.wait()
    cp_a.wait()
    cp_b.wait()

    ge8 = iota >> 3
    col = iota & 7
    acc = jnp.zeros((16,), jnp.float32)
    for k in range(NV):
        ridx = ge8 + (2 * k)
        if k < 64:
            p = plsc.load_gather(rows_a, [ridx, col])
        else:
            p = plsc.load_gather(rows_b, [ridx - 128, col])
        mrow = plsc.load_gather(mask_v, [ridx])
        t = tgt_v[pl.ds(k * 16, 16)]
        m = mrow * jnp.where(t != t, 0.0, 1.0)
        acc = acc + jnp.abs(p * m - t * m)

    accn = jnp.zeros((16,), jnp.float32)
    for k in range(PPAD // 16):
        accn = accn + mask_v[pl.ds(k * 16, 16)]

    part_v[0] = acc
    part_v[1] = accn
    pltpu.sync_copy(part_v, sacc_sh.at[w])
    plsc.subcore_barrier()

    @pl.when(w == 0)
    def _():
        pltpu.sync_copy(sacc_sh, acc_v)
        lacc = acc_v[0, 0]
        nacc = acc_v[0, 1]
        for r in range(1, NSUB):
            lacc = lacc + acc_v[r, 0]
            nacc = nacc + acc_v[r, 1]
        num = jnp.maximum(jnp.sum(nacc), 1.0)
        out_stage[...] = lacc
        hi = plsc.load_gather(out_stage, [(iota & 7) + 8])
        out_stage[...] = (lacc + hi) / num
        pltpu.sync_copy(out_stage.at[pl.ds(0, D)], out_hbm)


_call = functools.partial(
    pl.kernel,
    out_type=jax.ShapeDtypeStruct((D,), jnp.float32),
    mesh=plsc.VectorSubcoreMesh(core_axis_name="c", subcore_axis_name="s",
                                num_cores=1),
    compiler_params=pltpu.CompilerParams(needs_layout_passes=False,
                                         use_tc_tiling_on_sc=False,
                                         skip_device_barrier=True),
    scratch_types=[
        pltpu.VMEM((CHUNK,), jnp.int32),
        pltpu.VMEM((16,), jnp.int32),
        pltpu.VMEM((NSUB, 16), jnp.int32),
        pltpu.VMEM((16,), jnp.int32),
        pltpu.VMEM((PPAD,), jnp.int32),
        pltpu.VMEM((PPAD,), jnp.float32),
        pltpu.VMEM((PPW * D,), jnp.float32),
        pltpu.VMEM((128,), jnp.int32),
        pltpu.VMEM((128,), jnp.int32),
        pltpu.VMEM((128, D), jnp.float32),
        pltpu.VMEM((128, D), jnp.float32),
        pltpu.VMEM((2, 16), jnp.float32),
        pltpu.VMEM((NSUB, 2, 16), jnp.float32),
        pltpu.VMEM((16,), jnp.float32),
        pltpu.VMEM_SHARED((NSUB, 16), jnp.int32),
        pltpu.VMEM_SHARED((NSUB, 2, 16), jnp.float32),
        pltpu.SemaphoreType.DMA,
    ],
)(_body)


def kernel(output, mask, ind, target, batch_index):
    bidx = batch_index.astype(jnp.int32)
    pad = ((0, 0), (0, PPAD - PPW))
    ind16 = jnp.pad(ind.astype(jnp.int32).reshape(NSUB, PPW), pad)
    mask16 = jnp.pad(mask.reshape(NSUB, PPW).astype(jnp.float32), pad)
    tgt16 = target.reshape(NSUB, PPW * D)
    return _call(output, bidx, ind16, mask16, tgt16)


# speedup vs baseline: 3.9071x; 3.9037x over previous
import functools

import jax
import jax.numpy as jnp
from jax import lax
from jax.experimental import pallas as pl
from jax.experimental.pallas import tpu as pltpu
from jax.experimental.pallas import tpu_sc as plsc

N = 160000
B = 8
M = 500
D = 8
NSUB = 16
CHUNK = N // NSUB
PPW = (B * M) // NSUB
PPAD = 256
NV = PPW * D // 16
NVPAD = PPAD * D // 16


def _body(output_hbm, bidx_hbm, ind_hbm, mask_hbm, tgt_hbm, out_hbm,
          bi_v, cnt_v, all_v, starts_v, ind_v, mask_v, tgt_v,
          row_v, pred_v, part_v, acc_v, out_stage,
          sall_sh, sacc_sh, sem):
    w = lax.axis_index("s")
    iota = lax.iota(jnp.int32, 16)

    pltpu.sync_copy(bidx_hbm.at[pl.ds(w * CHUNK, CHUNK)], bi_v)
    pos = jnp.zeros((16,), jnp.int32)
    step = 8192
    while step:
        npos = pos + step
        probe = jnp.minimum(npos, CHUNK) - 1
        v = plsc.load_gather(bi_v, [probe])
        take = (npos <= CHUNK) & (v < iota)
        pos = jnp.where(take, npos, pos)
        step >>= 1
    cnt_v[...] = pos
    pltpu.sync_copy(cnt_v, sall_sh.at[w])
    plsc.subcore_barrier()
    pltpu.sync_copy(sall_sh, all_v)
    starts = all_v[0]
    for r in range(1, NSUB):
        starts = starts + all_v[r]
    starts_v[...] = starts

    b = w >> 1
    tgt_cp = pltpu.async_copy(tgt_hbm.at[w], tgt_v, sem)
    pltpu.sync_copy(ind_hbm.at[w], ind_v)
    pltpu.sync_copy(mask_hbm.at[w], mask_v)

    start_b = plsc.load_gather(starts_v, [jnp.full((16,), 0, jnp.int32) + b])
    for k in range(PPAD // 16):
        iv = ind_v[pl.ds(k * 16, 16)] + start_b
        row_v[pl.ds(k * 16, 16)] = jnp.minimum(jnp.maximum(iv, 0), N - 1)

    ge8 = iota >> 3
    col = iota & 7
    cps = []
    for k in range(NV):
        row2 = plsc.load_gather(row_v, [ge8 + (2 * k)])
        fidx = ((row2 >> 7) << 10) + (col << 7) + (row2 & 127)
        cps.append(pltpu.async_copy(output_hbm.at[fidx],
                                    pred_v.at[pl.ds(k * 16, 16)], sem))
    tgt_cp.wait()
    for cp in cps:
        cp.wait()

    acc = jnp.zeros((16,), jnp.float32)
    for k in range(NV):
        p = pred_v[pl.ds(k * 16, 16)]
        mrow = plsc.load_gather(mask_v, [ge8 + (2 * k)])
        t = tgt_v[pl.ds(k * 16, 16)]
        m = mrow * jnp.where(t != t, 0.0, 1.0)
        acc = acc + jnp.abs(p * m - t * m)

    accn = jnp.zeros((16,), jnp.float32)
    for k in range(PPAD // 16):
        accn = accn + mask_v[pl.ds(k * 16, 16)]

    part_v[0] = acc
    part_v[1] = accn
    pltpu.sync_copy(part_v, sacc_sh.at[w])
    plsc.subcore_barrier()

    @pl.when(w == 0)
    def _():
        pltpu.sync_copy(sacc_sh, acc_v)
        lacc = acc_v[0, 0]
        nacc = acc_v[0, 1]
        for r in range(1, NSUB):
            lacc = lacc + acc_v[r, 0]
            nacc = nacc + acc_v[r, 1]
        num = jnp.maximum(jnp.sum(nacc), 1.0)
        out_stage[...] = lacc
        hi = plsc.load_gather(out_stage, [(iota & 7) + 8])
        out_stage[...] = (lacc + hi) / num
        pltpu.sync_copy(out_stage.at[pl.ds(0, D)], out_hbm)


_call = functools.partial(
    pl.kernel,
    out_type=jax.ShapeDtypeStruct((D,), jnp.float32),
    mesh=plsc.VectorSubcoreMesh(core_axis_name="c", subcore_axis_name="s",
                                num_cores=1),
    compiler_params=pltpu.CompilerParams(needs_layout_passes=False,
                                         use_tc_tiling_on_sc=False,
                                         skip_device_barrier=True),
    scratch_types=[
        pltpu.VMEM((CHUNK,), jnp.int32),
        pltpu.VMEM((16,), jnp.int32),
        pltpu.VMEM((NSUB, 16), jnp.int32),
        pltpu.VMEM((16,), jnp.int32),
        pltpu.VMEM((PPAD,), jnp.int32),
        pltpu.VMEM((PPAD,), jnp.float32),
        pltpu.VMEM((PPW * D,), jnp.float32),
        pltpu.VMEM((PPAD,), jnp.int32),
        pltpu.VMEM((PPW * D,), jnp.float32),
        pltpu.VMEM((2, 16), jnp.float32),
        pltpu.VMEM((NSUB, 2, 16), jnp.float32),
        pltpu.VMEM((16,), jnp.float32),
        pltpu.VMEM_SHARED((NSUB, 16), jnp.int32),
        pltpu.VMEM_SHARED((NSUB, 2, 16), jnp.float32),
        pltpu.SemaphoreType.DMA,
    ],
)(_body)


def kernel(output, mask, ind, target, batch_index):
    bidx = batch_index.astype(jnp.int32)
    out_lin = output.T.reshape(D, N // 128, 128).transpose(1, 0, 2).reshape(-1)
    pad = ((0, 0), (0, PPAD - PPW))
    ind16 = jnp.pad(ind.astype(jnp.int32).reshape(NSUB, PPW), pad)
    mask16 = jnp.pad(mask.reshape(NSUB, PPW).astype(jnp.float32), pad)
    tgt16 = target.reshape(NSUB, PPW * D)
    return _call(out_lin, bidx, ind16, mask16, tgt16)


# speedup vs baseline: 4.2171x; 1.0793x over previous
import functools

import jax
import jax.numpy as jnp
from jax import lax
from jax.experimental import pallas as pl
from jax.experimental.pallas import tpu as pltpu
from jax.experimental.pallas import tpu_sc as plsc

N = 160000
B = 8
M = 500
D = 8
NSUB = 16
CHUNK = N // NSUB
PPW = (B * M) // NSUB
PPAD = 256
NV = PPW * D // 16
NVPAD = PPAD * D // 16


def _body(output_hbm, bidx_hbm, pk_hbm, tgt_hbm, out_hbm,
          bi_v, cnt_v, all_v, starts_v, pk_v, tgt_v,
          row_v, pred_v, part_v, acc_v, out_stage,
          sall_sh, sacc_sh, sem, sem_pk, sem_tgt):
    w = lax.axis_index("s")
    iota = lax.iota(jnp.int32, 16)
    b = w >> 1
    h = w & 1

    tgt_cp = pltpu.async_copy(tgt_hbm.at[b, pl.ds(h * PPW * D, PPW * D)],
                              tgt_v, sem_tgt)
    pk_cp = pltpu.async_copy(pk_hbm.at[w], pk_v, sem_pk)

    pltpu.sync_copy(bidx_hbm.at[pl.ds(w * CHUNK, CHUNK)], bi_v)
    pos = jnp.zeros((16,), jnp.int32)
    step = 8192
    while step:
        npos = pos + step
        probe = jnp.minimum(npos, CHUNK) - 1
        v = plsc.load_gather(bi_v, [probe])
        take = (npos <= CHUNK) & (v < iota)
        pos = jnp.where(take, npos, pos)
        step >>= 1
    cnt_v[...] = pos
    pltpu.sync_copy(cnt_v, sall_sh.at[w])
    plsc.subcore_barrier()
    pltpu.sync_copy(sall_sh, all_v)
    starts = all_v[0]
    for r in range(1, NSUB):
        starts = starts + all_v[r]
    starts_v[...] = starts

    pk_cp.wait()
    start_b = plsc.load_gather(starts_v, [jnp.full((16,), 0, jnp.int32) + b])
    for k in range(PPAD // 16):
        iv = (pk_v[pl.ds(k * 16, 16)] & 0xFFFFF) + start_b
        row_v[pl.ds(k * 16, 16)] = jnp.minimum(jnp.maximum(iv, 0), N - 1)

    ge8 = iota >> 3
    col = iota & 7
    cps = []
    for k in range(NV):
        row2 = plsc.load_gather(row_v, [ge8 + (2 * k)])
        fidx = ((row2 >> 7) << 10) + (col << 7) + (row2 & 127)
        cps.append(pltpu.async_copy(output_hbm.at[fidx],
                                    pred_v.at[pl.ds(k * 16, 16)], sem))
    tgt_cp.wait()
    for cp in cps:
        cp.wait()

    acc = jnp.zeros((16,), jnp.float32)
    for k in range(NV):
        p = pred_v[pl.ds(k * 16, 16)]
        g = plsc.load_gather(pk_v, [ge8 + (2 * k)])
        t = tgt_v[pl.ds(k * 16, 16)]
        m = (g >> 20).astype(jnp.float32) * jnp.where(t != t, 0.0, 1.0)
        acc = acc + jnp.abs(p * m - t * m)

    accn = jnp.zeros((16,), jnp.float32)
    for k in range(PPAD // 16):
        accn = accn + (pk_v[pl.ds(k * 16, 16)] >> 20).astype(jnp.float32)

    part_v[0] = acc
    part_v[1] = accn
    pltpu.sync_copy(part_v, sacc_sh.at[w])
    plsc.subcore_barrier()

    @pl.when(w == 0)
    def _():
        pltpu.sync_copy(sacc_sh, acc_v)
        lacc = acc_v[0, 0]
        nacc = acc_v[0, 1]
        for r in range(1, NSUB):
            lacc = lacc + acc_v[r, 0]
            nacc = nacc + acc_v[r, 1]
        num = jnp.maximum(jnp.sum(nacc), 1.0)
        out_stage[...] = lacc
        hi = plsc.load_gather(out_stage, [(iota & 7) + 8])
        out_stage[...] = (lacc + hi) / num
        pltpu.sync_copy(out_stage.at[pl.ds(0, D)], out_hbm)


_call = functools.partial(
    pl.kernel,
    out_type=jax.ShapeDtypeStruct((D,), jnp.float32),
    mesh=plsc.VectorSubcoreMesh(core_axis_name="c", subcore_axis_name="s",
                                num_cores=1),
    compiler_params=pltpu.CompilerParams(needs_layout_passes=False,
                                         use_tc_tiling_on_sc=False,
                                         skip_device_barrier=True),
    scratch_types=[
        pltpu.VMEM((CHUNK,), jnp.int32),
        pltpu.VMEM((16,), jnp.int32),
        pltpu.VMEM((NSUB, 16), jnp.int32),
        pltpu.VMEM((16,), jnp.int32),
        pltpu.VMEM((PPAD,), jnp.int32),
        pltpu.VMEM((PPW * D,), jnp.float32),
        pltpu.VMEM((PPAD,), jnp.int32),
        pltpu.VMEM((PPW * D,), jnp.float32),
        pltpu.VMEM((2, 16), jnp.float32),
        pltpu.VMEM((NSUB, 2, 16), jnp.float32),
        pltpu.VMEM((16,), jnp.float32),
        pltpu.VMEM_SHARED((NSUB, 16), jnp.int32),
        pltpu.VMEM_SHARED((NSUB, 2, 16), jnp.float32),
        pltpu.SemaphoreType.DMA,
        pltpu.SemaphoreType.DMA,
        pltpu.SemaphoreType.DMA,
    ],
)(_body)


def kernel(output, mask, ind, target, batch_index):
    bidx = batch_index.astype(jnp.int32)
    out_lin = output.T.reshape(D, N // 128, 128).transpose(1, 0, 2).reshape(-1)
    pad = ((0, 0), (0, PPAD - PPW))
    packed = ind.astype(jnp.int32) | (mask.astype(jnp.int32) << 20)
    pk16 = jnp.pad(packed.reshape(NSUB, PPW), pad)
    tgt8 = target.reshape(B, M * D)
    return _call(out_lin, bidx, pk16, tgt8)


# speedup vs baseline: 4.7067x; 1.1161x over previous
import functools

import jax
import jax.numpy as jnp
from jax import lax
from jax.experimental import pallas as pl
from jax.experimental.pallas import tpu as pltpu
from jax.experimental.pallas import tpu_sc as plsc

N = 160000
B = 8
M = 500
D = 8
NSUB = 16
CHUNK = N // NSUB
PPW = (B * M) // NSUB
PPAD = 256
NV = PPW * D // 16
NVPAD = PPAD * D // 16


def _body(output_hbm, bidx_hbm, pk_hbm, tgt_hbm, out_hbm,
          bi_v, cnt_v, all_v, starts_v, pk_v, tgt_v,
          row_v, pred_v, part_v, acc_v, out_stage,
          sall_sh, sacc_sh, sem, sem_pk, sem_tgt):
    w = lax.axis_index("s")
    iota = lax.iota(jnp.int32, 16)
    b = w >> 1
    h = w & 1

    tgt_cp = pltpu.async_copy(tgt_hbm.at[b, pl.ds(h * PPW * D, PPW * D)],
                              tgt_v, sem_tgt)
    pk_cp = pltpu.async_copy(pk_hbm.at[w], pk_v, sem_pk)

    pltpu.sync_copy(bidx_hbm.at[pl.ds(w * CHUNK, CHUNK)], bi_v)
    pos = jnp.zeros((16,), jnp.int32)
    step = 8192
    while step:
        npos = pos + step
        probe = jnp.minimum(npos, CHUNK) - 1
        v = plsc.load_gather(bi_v, [probe])
        take = (npos <= CHUNK) & (v < iota)
        pos = jnp.where(take, npos, pos)
        step >>= 1
    cnt_v[...] = pos
    pltpu.sync_copy(cnt_v, sall_sh.at[w])
    plsc.subcore_barrier()
    pltpu.sync_copy(sall_sh, all_v)

    def _sum_starts(r, a):
        return a + all_v[r]

    starts_v[...] = lax.fori_loop(1, NSUB, _sum_starts, all_v[0])

    pk_cp.wait()
    start_b = plsc.load_gather(starts_v, [jnp.full((16,), 0, jnp.int32) + b])

    def _rows(k, a):
        pk = pk_v[pl.ds(k * 16, 16)]
        iv = (pk & 0xFFFFF) + start_b
        row_v[pl.ds(k * 16, 16)] = jnp.minimum(jnp.maximum(iv, 0), N - 1)
        return a + (pk >> 20).astype(jnp.float32)

    accn = lax.fori_loop(0, PPAD // 16, _rows,
                         jnp.zeros((16,), jnp.float32))

    ge8 = iota >> 3
    col = iota & 7

    def _gather(k, a):
        row2 = plsc.load_gather(row_v, [ge8 + 2 * k])
        fidx = ((row2 >> 7) << 10) + (col << 7) + (row2 & 127)
        pltpu.async_copy(output_hbm.at[fidx],
                         pred_v.at[pl.ds(k * 16, 16)], sem)
        return a

    lax.fori_loop(0, NV, _gather, 0)
    tgt_cp.wait()
    pltpu.make_async_copy(output_hbm.at[pl.ds(0, PPW * D)], pred_v, sem).wait()

    def _loss(k, a):
        p = pred_v[pl.ds(k * 16, 16)]
        g = plsc.load_gather(pk_v, [ge8 + 2 * k])
        t = tgt_v[pl.ds(k * 16, 16)]
        m = (g >> 20).astype(jnp.float32) * jnp.where(t != t, 0.0, 1.0)
        return a + jnp.abs(p * m - t * m)

    acc = lax.fori_loop(0, NV, _loss, jnp.zeros((16,), jnp.float32))

    part_v[0] = acc
    part_v[1] = accn
    pltpu.sync_copy(part_v, sacc_sh.at[w])
    plsc.subcore_barrier()

    @pl.when(w == 0)
    def _():
        pltpu.sync_copy(sacc_sh, acc_v)

        def _comb(r, la_na):
            la, na = la_na
            return la + acc_v[r, 0], na + acc_v[r, 1]

        lacc, nacc = lax.fori_loop(1, NSUB, _comb, (acc_v[0, 0], acc_v[0, 1]))
        num = jnp.maximum(jnp.sum(nacc), 1.0)
        out_stage[...] = lacc
        hi = plsc.load_gather(out_stage, [(iota & 7) + 8])
        out_stage[...] = (lacc + hi) / num
        pltpu.sync_copy(out_stage.at[pl.ds(0, D)], out_hbm)


_call = functools.partial(
    pl.kernel,
    out_type=jax.ShapeDtypeStruct((D,), jnp.float32),
    mesh=plsc.VectorSubcoreMesh(core_axis_name="c", subcore_axis_name="s",
                                num_cores=1),
    compiler_params=pltpu.CompilerParams(needs_layout_passes=False,
                                         use_tc_tiling_on_sc=False,
                                         skip_device_barrier=True),
    scratch_types=[
        pltpu.VMEM((CHUNK,), jnp.int32),
        pltpu.VMEM((16,), jnp.int32),
        pltpu.VMEM((NSUB, 16), jnp.int32),
        pltpu.VMEM((16,), jnp.int32),
        pltpu.VMEM((PPAD,), jnp.int32),
        pltpu.VMEM((PPW * D,), jnp.float32),
        pltpu.VMEM((PPAD,), jnp.int32),
        pltpu.VMEM((PPW * D,), jnp.float32),
        pltpu.VMEM((2, 16), jnp.float32),
        pltpu.VMEM((NSUB, 2, 16), jnp.float32),
        pltpu.VMEM((16,), jnp.float32),
        pltpu.VMEM_SHARED((NSUB, 16), jnp.int32),
        pltpu.VMEM_SHARED((NSUB, 2, 16), jnp.float32),
        pltpu.SemaphoreType.DMA,
        pltpu.SemaphoreType.DMA,
        pltpu.SemaphoreType.DMA,
    ],
)(_body)


def kernel(output, mask, ind, target, batch_index):
    bidx = batch_index.astype(jnp.int32)
    out_lin = output.T.reshape(D, N // 128, 128).transpose(1, 0, 2).reshape(-1)
    pad = ((0, 0), (0, PPAD - PPW))
    packed = ind.astype(jnp.int32) | (mask.astype(jnp.int32) << 20)
    pk16 = jnp.pad(packed.reshape(NSUB, PPW), pad)
    tgt8 = target.reshape(B, M * D)
    return _call(out_lin, bidx, pk16, tgt8)


# speedup vs baseline: 4.7214x; 1.0031x over previous
import functools

import jax
import jax.numpy as jnp
from jax import lax
from jax.experimental import pallas as pl
from jax.experimental.pallas import tpu as pltpu
from jax.experimental.pallas import tpu_sc as plsc

N = 160000
B = 8
M = 500
D = 8
NSUB = 16
CHUNK = N // NSUB
PPW = (B * M) // NSUB
PPAD = 256
NV = PPW * D // 16
NVPAD = PPAD * D // 16


def _body(output_hbm, bidx_hbm, pk_hbm, tgt_hbm, out_hbm,
          bi_v, cnt_v, all_v, starts_v, pk_v, tgt_v,
          row_v, pred_v, part_v, acc_v, out_stage,
          sall_sh, sacc_sh, sem, sem_pk, sem_tgt):
    w = lax.axis_index("s")
    iota = lax.iota(jnp.int32, 16)
    b = w >> 1
    h = w & 1

    tgt_cp = pltpu.async_copy(tgt_hbm.at[b, pl.ds(h * PPW * D, PPW * D)],
                              tgt_v, sem_tgt)
    pk_cp = pltpu.async_copy(pk_hbm.at[w], pk_v, sem_pk)

    import jax as _jax
    with _jax.named_scope("phA_dma"):
        pltpu.sync_copy(bidx_hbm.at[pl.ds(w * CHUNK, CHUNK)], bi_v)
    pos = jnp.zeros((16,), jnp.int32)
    step = 8192
    while step:
        npos = pos + step
        probe = jnp.minimum(npos, CHUNK) - 1
        v = plsc.load_gather(bi_v, [probe])
        take = (npos <= CHUNK) & (v < iota)
        pos = jnp.where(take, npos, pos)
        step >>= 1
    cnt_v[...] = pos
    with _jax.named_scope("phA_comb"):
        pltpu.sync_copy(cnt_v, sall_sh.at[w])
        plsc.subcore_barrier()
        pltpu.sync_copy(sall_sh, all_v)

    def _sum_starts(r, a):
        return a + all_v[r]

    starts_v[...] = lax.fori_loop(1, NSUB, _sum_starts, all_v[0])

    pk_cp.wait()
    start_b = plsc.load_gather(starts_v, [jnp.full((16,), 0, jnp.int32) + b])

    def _rows(k, a):
        pk = pk_v[pl.ds(k * 16, 16)]
        iv = (pk & 0xFFFFF) + start_b
        row_v[pl.ds(k * 16, 16)] = jnp.minimum(jnp.maximum(iv, 0), N - 1)
        return a + (pk >> 20).astype(jnp.float32)

    accn = lax.fori_loop(0, PPAD // 16, _rows,
                         jnp.zeros((16,), jnp.float32))

    ge8 = iota >> 3
    col = iota & 7

    def _gather(k, a):
        row2 = plsc.load_gather(row_v, [ge8 + 2 * k])
        fidx = ((row2 >> 7) << 10) + (col << 7) + (row2 & 127)
        pltpu.async_copy(output_hbm.at[fidx],
                         pred_v.at[pl.ds(k * 16, 16)], sem)
        return a

    with _jax.named_scope("gather_fire"):
        lax.fori_loop(0, NV, _gather, 0)
    tgt_cp.wait()
    with _jax.named_scope("gather_drain"):
        pltpu.make_async_copy(output_hbm.at[pl.ds(0, PPW * D)], pred_v, sem).wait()

    def _loss(k, a):
        p = pred_v[pl.ds(k * 16, 16)]
        g = plsc.load_gather(pk_v, [ge8 + 2 * k])
        t = tgt_v[pl.ds(k * 16, 16)]
        m = (g >> 20).astype(jnp.float32) * jnp.where(t != t, 0.0, 1.0)
        return a + jnp.abs(p * m - t * m)

    with _jax.named_scope("loss"):
        acc = lax.fori_loop(0, NV, _loss, jnp.zeros((16,), jnp.float32))

    with _jax.named_scope("fin_comb"):
        part_v[0] = acc
        part_v[1] = accn
        pltpu.sync_copy(part_v, sacc_sh.at[w])
        plsc.subcore_barrier()

    @pl.when(w == 0)
    def _():
        pltpu.sync_copy(sacc_sh, acc_v)

        def _comb(r, la_na):
            la, na = la_na
            return la + acc_v[r, 0], na + acc_v[r, 1]

        lacc, nacc = lax.fori_loop(1, NSUB, _comb, (acc_v[0, 0], acc_v[0, 1]))
        num = jnp.maximum(jnp.sum(nacc), 1.0)
        out_stage[...] = lacc
        hi = plsc.load_gather(out_stage, [(iota & 7) + 8])
        out_stage[...] = (lacc + hi) / num
        pltpu.sync_copy(out_stage.at[pl.ds(0, D)], out_hbm)


_call = functools.partial(
    pl.kernel,
    out_type=jax.ShapeDtypeStruct((D,), jnp.float32),
    mesh=plsc.VectorSubcoreMesh(core_axis_name="c", subcore_axis_name="s",
                                num_cores=1),
    compiler_params=pltpu.CompilerParams(needs_layout_passes=False,
                                         use_tc_tiling_on_sc=False,
                                         skip_device_barrier=True),
    scratch_types=[
        pltpu.VMEM((CHUNK,), jnp.int32),
        pltpu.VMEM((16,), jnp.int32),
        pltpu.VMEM((NSUB, 16), jnp.int32),
        pltpu.VMEM((16,), jnp.int32),
        pltpu.VMEM((PPAD,), jnp.int32),
        pltpu.VMEM((PPW * D,), jnp.float32),
        pltpu.VMEM((PPAD,), jnp.int32),
        pltpu.VMEM((PPW * D,), jnp.float32),
        pltpu.VMEM((2, 16), jnp.float32),
        pltpu.VMEM((NSUB, 2, 16), jnp.float32),
        pltpu.VMEM((16,), jnp.float32),
        pltpu.VMEM_SHARED((NSUB, 16), jnp.int32),
        pltpu.VMEM_SHARED((NSUB, 2, 16), jnp.float32),
        pltpu.SemaphoreType.DMA,
        pltpu.SemaphoreType.DMA,
        pltpu.SemaphoreType.DMA,
    ],
)(_body)


def kernel(output, mask, ind, target, batch_index):
    bidx = batch_index.astype(jnp.int32)
    out_lin = output.T.reshape(D, N // 128, 128).transpose(1, 0, 2).reshape(-1)
    pad = ((0, 0), (0, PPAD - PPW))
    packed = ind.astype(jnp.int32) | (mask.astype(jnp.int32) << 20)
    pk16 = jnp.pad(packed.reshape(NSUB, PPW), pad)
    tgt8 = target.reshape(B, M * D)
    return _call(out_lin, bidx, pk16, tgt8)
